# baseline (speedup 1.0000x reference)
import numpy as np

import jax
import jax.numpy as jnp
from jax.experimental import pallas as pl
from jax.experimental.pallas import tpu as pltpu

_TB = 128


def _fused_kernel(x_ref, b1_ref, b2_ref, f1_ref, f2_ref,
                  bb1_ref, bb2_ref, fb1_ref, fb2_ref, o_ref):
    tb = x_ref.shape[1]

    acc1 = [[None, None] for _ in range(4)]
    for r in range(4):
        a = x_ref[r].reshape(tb * 7, 160)
        for p in range(2):
            acc1[r][p] = jnp.dot(a, b1_ref[p],
                                 preferred_element_type=jnp.float32)
    hpe = jnp.maximum(
        jnp.maximum(jnp.maximum(acc1[0][0], acc1[0][1]),
                    jnp.maximum(acc1[1][0], acc1[1][1])) + bb1_ref[...], 0.0
    ).astype(jnp.bfloat16).reshape(tb, 7, 280)
    hpo = jnp.maximum(
        jnp.maximum(jnp.maximum(acc1[2][0], acc1[2][1]),
                    jnp.maximum(acc1[3][0], acc1[3][1])) + bb1_ref[...], 0.0
    ).astype(jnp.bfloat16).reshape(tb, 7, 280)

    acc2 = [[None, None] for _ in range(2)]
    for q in range(2):
        for kh in range(5):
            src = hpe if (q + kh) % 2 == 0 else hpo
            t = (q + kh) // 2
            a = src[:, t:t + 5, :].reshape(tb * 5, 280)
            for p in range(2):
                d = jnp.dot(a, b2_ref[2 * kh + p],
                            preferred_element_type=jnp.float32)
                acc2[q][p] = d if acc2[q][p] is None else acc2[q][p] + d
    hp2 = jnp.maximum(
        jnp.maximum(jnp.maximum(acc2[0][0], acc2[0][1]),
                    jnp.maximum(acc2[1][0], acc2[1][1])) + bb2_ref[...], 0.0
    ).astype(jnp.bfloat16).reshape(tb, 5, 250)

    acc = None
    for h in range(5):
        d = jnp.dot(hp2[:, h, :], f1_ref[h], preferred_element_type=jnp.float32)
        acc = d if acc is None else acc + d
    hfc = jnp.maximum(acc + fb1_ref[...], 0.0).astype(jnp.bfloat16)
    o_ref[...] = jnp.dot(hfc, f2_ref[...],
                         preferred_element_type=jnp.float32) + fb2_ref[...]


def _band_select(ow_count, w_count):
    s = np.zeros((5, 2, w_count, ow_count), np.float32)
    for kw in range(5):
        for p in range(2):
            for ow2 in range(ow_count):
                s[kw, p, 2 * ow2 + p + kw, ow2] = 1.0
    return s


def _build_tables(conv1_w, conv1_b, conv2_w, conv2_b,
                  fc1_w, fc1_b, fc2_w, fc2_b):
    f32 = jnp.float32
    bf16 = jnp.bfloat16
    w1 = conv1_w[:, 0].transpose(1, 2, 0).astype(f32)
    w2 = conv2_w.transpose(2, 3, 1, 0).astype(f32)

    s1 = jnp.asarray(_band_select(14, 32))
    b1 = jnp.einsum('akc,kpwm->pawmc', w1, s1)
    b1 = b1.reshape(2, 160, 280).astype(bf16)

    s2 = jnp.asarray(_band_select(5, 14))
    b2 = jnp.einsum('akco,kpwm->apwcmo', w2, s2)
    b2 = b2.reshape(10, 280, 250).astype(bf16)

    t1 = fc1_w.astype(f32).reshape(320, 50, 5, 5).transpose(2, 3, 1, 0)
    f1 = t1.reshape(5, 250, 320).astype(bf16)

    f2 = jnp.pad(fc2_w.astype(f32).T, ((0, 0), (0, 118))).astype(bf16)

    bb1 = jnp.tile(conv1_b.astype(f32), 14)[None]
    bb2 = jnp.tile(conv2_b.astype(f32), 5)[None]
    fb1 = fc1_b.astype(f32)[None]
    fb2 = jnp.pad(fc2_b.astype(f32), (0, 118))[None]
    return b1, b2, f1, f2, bb1, bb2, fb1, fb2


def kernel(x, conv1_w, conv1_b, conv2_w, conv2_b, fc1_w, fc1_b, fc2_w, fc2_b):
    tables = _build_tables(conv1_w, conv1_b, conv2_w, conv2_b,
                           fc1_w, fc1_b, fc2_w, fc2_b)
    B = x.shape[0]
    xpb = jnp.pad(x.reshape(B, 28, 28).astype(jnp.float32),
                  ((0, 0), (2, 2), (2, 2))).astype(jnp.bfloat16)
    planes = []
    for r in range(4):
        pieces = [xpb[:, r + kh::4, :][:, :7, :] for kh in range(5)]
        planes.append(jnp.concatenate(pieces, axis=-1))
    xcat = jnp.stack(planes, axis=0)
    out = pl.pallas_call(
        _fused_kernel,
        out_shape=jax.ShapeDtypeStruct((B, 128), jnp.float32),
        grid=(B // _TB,),
        in_specs=[
            pl.BlockSpec((4, _TB, 7, 160), lambda i: (0, i, 0, 0)),
            pl.BlockSpec((2, 160, 280), lambda i: (0, 0, 0)),
            pl.BlockSpec((10, 280, 250), lambda i: (0, 0, 0)),
            pl.BlockSpec((5, 250, 320), lambda i: (0, 0, 0)),
            pl.BlockSpec((320, 128), lambda i: (0, 0)),
            pl.BlockSpec((1, 280), lambda i: (0, 0)),
            pl.BlockSpec((1, 250), lambda i: (0, 0)),
            pl.BlockSpec((1, 320), lambda i: (0, 0)),
            pl.BlockSpec((1, 128), lambda i: (0, 0)),
        ],
        out_specs=pl.BlockSpec((_TB, 128), lambda i: (i, 0)),
        compiler_params=pltpu.CompilerParams(dimension_semantics=("parallel",)),
    )(xcat, *tables)
    return out[:, :10]


# speedup vs baseline: 4.2160x; 4.2160x over previous
import numpy as np

import jax
import jax.numpy as jnp
from jax.experimental import pallas as pl
from jax.experimental.pallas import tpu as pltpu

_TB = 128


def _fused_kernel(x_ref, b1_ref, b2_ref, f1_ref, f2_ref,
                  bb1_ref, bb2_ref, fb1_ref, fb2_ref, o_ref):
    tb = x_ref.shape[0]

    xv = x_ref[...].reshape(tb * 8, 128)
    xs = jnp.roll(xv, -1, axis=0)
    a1 = jnp.concatenate([xv, xs], axis=-1).astype(jnp.bfloat16)

    acc1 = [[None, None] for _ in range(4)]
    for r in range(4):
        for p in range(2):
            acc1[r][p] = jnp.dot(a1, b1_ref[2 * r + p],
                                 preferred_element_type=jnp.float32)
    hpe = jnp.maximum(
        jnp.maximum(jnp.maximum(acc1[0][0], acc1[0][1]),
                    jnp.maximum(acc1[1][0], acc1[1][1])) + bb1_ref[...], 0.0
    ).astype(jnp.bfloat16).reshape(tb, 8, 280)
    hpo = jnp.maximum(
        jnp.maximum(jnp.maximum(acc1[2][0], acc1[2][1]),
                    jnp.maximum(acc1[3][0], acc1[3][1])) + bb1_ref[...], 0.0
    ).astype(jnp.bfloat16).reshape(tb, 8, 280)

    acc2 = [[None, None] for _ in range(2)]
    for q in range(2):
        for kh in range(5):
            src = hpe if (q + kh) % 2 == 0 else hpo
            t = (q + kh) // 2
            a = src[:, t:t + 5, :].reshape(tb * 5, 280)
            for p in range(2):
                d = jnp.dot(a, b2_ref[2 * kh + p],
                            preferred_element_type=jnp.float32)
                acc2[q][p] = d if acc2[q][p] is None else acc2[q][p] + d
    hp2 = jnp.maximum(
        jnp.maximum(jnp.maximum(acc2[0][0], acc2[0][1]),
                    jnp.maximum(acc2[1][0], acc2[1][1])) + bb2_ref[...], 0.0
    ).astype(jnp.bfloat16).reshape(tb, 5, 250)

    acc = None
    for h in range(5):
        d = jnp.dot(hp2[:, h, :], f1_ref[h], preferred_element_type=jnp.float32)
        acc = d if acc is None else acc + d
    hfc = jnp.maximum(acc + fb1_ref[...], 0.0).astype(jnp.bfloat16)
    o_ref[...] = jnp.dot(hfc, f2_ref[...],
                         preferred_element_type=jnp.float32) + fb2_ref[...]


def _band_select(ow_count, w_count):
    s = np.zeros((5, 2, w_count, ow_count), np.float32)
    for kw in range(5):
        for p in range(2):
            for ow2 in range(ow_count):
                s[kw, p, 2 * ow2 + p + kw, ow2] = 1.0
    return s


def _conv1_row_index():
    idx = np.full((4, 256), 160, np.int32)
    for r in range(4):
        for s in range(2):
            for t in range(4):
                kh = 4 * s + t - r
                if 0 <= kh <= 4:
                    for wp in range(32):
                        idx[r, s * 128 + t * 32 + wp] = kh * 32 + wp
    return idx


def _build_tables(conv1_w, conv1_b, conv2_w, conv2_b,
                  fc1_w, fc1_b, fc2_w, fc2_b):
    f32 = jnp.float32
    bf16 = jnp.bfloat16
    w1 = conv1_w[:, 0].transpose(1, 2, 0).astype(f32)
    w2 = conv2_w.transpose(2, 3, 1, 0).astype(f32)

    s1 = jnp.asarray(_band_select(14, 32))
    b1k = jnp.einsum('akc,kpwm->pawmc', w1, s1)
    b1k = b1k.reshape(2, 160, 280)
    b1aug = jnp.concatenate([b1k, jnp.zeros((2, 1, 280), f32)], axis=1)
    idx = jnp.asarray(_conv1_row_index())
    b1 = b1aug[:, idx, :].transpose(1, 0, 2, 3)
    b1 = b1.reshape(8, 256, 280).astype(bf16)

    s2 = jnp.asarray(_band_select(5, 14))
    b2 = jnp.einsum('akco,kpwm->apwcmo', w2, s2)
    b2 = b2.reshape(10, 280, 250).astype(bf16)

    t1 = fc1_w.astype(f32).reshape(320, 50, 5, 5).transpose(2, 3, 1, 0)
    f1 = t1.reshape(5, 250, 320).astype(bf16)

    f2 = jnp.pad(fc2_w.astype(f32).T, ((0, 0), (0, 118))).astype(bf16)

    bb1 = jnp.tile(conv1_b.astype(f32), 14)[None]
    bb2 = jnp.tile(conv2_b.astype(f32), 5)[None]
    fb1 = fc1_b.astype(f32)[None]
    fb2 = jnp.pad(fc2_b.astype(f32), (0, 118))[None]
    return b1, b2, f1, f2, bb1, bb2, fb1, fb2


def kernel(x, conv1_w, conv1_b, conv2_w, conv2_b, fc1_w, fc1_b, fc2_w, fc2_b):
    tables = _build_tables(conv1_w, conv1_b, conv2_w, conv2_b,
                           fc1_w, fc1_b, fc2_w, fc2_b)
    B = x.shape[0]
    x4 = jnp.pad(x.reshape(B, 28, 28).astype(jnp.float32),
                 ((0, 0), (2, 2), (2, 2))).reshape(B, 8, 128)
    out = pl.pallas_call(
        _fused_kernel,
        out_shape=jax.ShapeDtypeStruct((B, 128), jnp.float32),
        grid=(B // _TB,),
        in_specs=[
            pl.BlockSpec((_TB, 8, 128), lambda i: (i, 0, 0)),
            pl.BlockSpec((8, 256, 280), lambda i: (0, 0, 0)),
            pl.BlockSpec((10, 280, 250), lambda i: (0, 0, 0)),
            pl.BlockSpec((5, 250, 320), lambda i: (0, 0, 0)),
            pl.BlockSpec((320, 128), lambda i: (0, 0)),
            pl.BlockSpec((1, 280), lambda i: (0, 0)),
            pl.BlockSpec((1, 250), lambda i: (0, 0)),
            pl.BlockSpec((1, 320), lambda i: (0, 0)),
            pl.BlockSpec((1, 128), lambda i: (0, 0)),
        ],
        out_specs=pl.BlockSpec((_TB, 128), lambda i: (i, 0)),
        compiler_params=pltpu.CompilerParams(dimension_semantics=("parallel",)),
    )(x4, *tables)
    return out[:, :10]


# speedup vs baseline: 4.3008x; 1.0201x over previous
import numpy as np

import jax
import jax.numpy as jnp
from jax.experimental import pallas as pl
from jax.experimental.pallas import tpu as pltpu

_TB = 256


def _fused_kernel(x_ref, b1_ref, b2_ref, f1_ref, f2_ref,
                  bb1_ref, bb2_ref, fb1_ref, fb2_ref, o_ref):
    tb = x_ref.shape[0]

    xv = x_ref[...].reshape(tb * 8, 128)
    xs = jnp.roll(xv, -1, axis=0)
    a1 = jnp.concatenate([xv, xs], axis=-1).astype(jnp.bfloat16)

    acc1 = [[None, None] for _ in range(4)]
    for r in range(4):
        for p in range(2):
            acc1[r][p] = jnp.dot(a1, b1_ref[2 * r + p],
                                 preferred_element_type=jnp.float32)
    hpe = jnp.maximum(
        jnp.maximum(jnp.maximum(acc1[0][0], acc1[0][1]),
                    jnp.maximum(acc1[1][0], acc1[1][1])) + bb1_ref[...], 0.0
    ).astype(jnp.bfloat16).reshape(tb, 8, 280)
    hpo = jnp.maximum(
        jnp.maximum(jnp.maximum(acc1[2][0], acc1[2][1]),
                    jnp.maximum(acc1[3][0], acc1[3][1])) + bb1_ref[...], 0.0
    ).astype(jnp.bfloat16).reshape(tb, 8, 280)

    acc2 = [[None, None] for _ in range(2)]
    for q in range(2):
        for kh in range(5):
            src = hpe if (q + kh) % 2 == 0 else hpo
            t = (q + kh) // 2
            a = src[:, t:t + 5, :].reshape(tb * 5, 280)
            for p in range(2):
                d = jnp.dot(a, b2_ref[2 * kh + p],
                            preferred_element_type=jnp.float32)
                acc2[q][p] = d if acc2[q][p] is None else acc2[q][p] + d
    hp2 = jnp.maximum(
        jnp.maximum(jnp.maximum(acc2[0][0], acc2[0][1]),
                    jnp.maximum(acc2[1][0], acc2[1][1])) + bb2_ref[...], 0.0
    ).astype(jnp.bfloat16).reshape(tb, 5, 250)

    acc = None
    for h in range(5):
        d = jnp.dot(hp2[:, h, :], f1_ref[h], preferred_element_type=jnp.float32)
        acc = d if acc is None else acc + d
    hfc = jnp.maximum(acc + fb1_ref[...], 0.0).astype(jnp.bfloat16)
    o_ref[...] = (jnp.dot(hfc, f2_ref[...],
                          preferred_element_type=jnp.float32)
                  + fb2_ref[...])[:, :10]


def _band_select(ow_count, w_count):
    s = np.zeros((5, 2, w_count, ow_count), np.float32)
    for kw in range(5):
        for p in range(2):
            for ow2 in range(ow_count):
                s[kw, p, 2 * ow2 + p + kw, ow2] = 1.0
    return s


def _conv1_row_index():
    idx = np.full((4, 256), 160, np.int32)
    for r in range(4):
        for s in range(2):
            for t in range(4):
                kh = 4 * s + t - r
                if 0 <= kh <= 4:
                    for wp in range(32):
                        idx[r, s * 128 + t * 32 + wp] = kh * 32 + wp
    return idx


def _build_tables(conv1_w, conv1_b, conv2_w, conv2_b,
                  fc1_w, fc1_b, fc2_w, fc2_b):
    f32 = jnp.float32
    bf16 = jnp.bfloat16
    w1 = conv1_w[:, 0].transpose(1, 2, 0).astype(f32)
    w2 = conv2_w.transpose(2, 3, 1, 0).astype(f32)

    s1 = jnp.asarray(_band_select(14, 32))
    b1k = jnp.einsum('akc,kpwm->pawmc', w1, s1)
    b1k = b1k.reshape(2, 160, 280)
    b1aug = jnp.concatenate([b1k, jnp.zeros((2, 1, 280), f32)], axis=1)
    idx = jnp.asarray(_conv1_row_index())
    b1 = b1aug[:, idx, :].transpose(1, 0, 2, 3)
    b1 = b1.reshape(8, 256, 280).astype(bf16)

    s2 = jnp.asarray(_band_select(5, 14))
    b2 = jnp.einsum('akco,kpwm->apwcmo', w2, s2)
    b2 = b2.reshape(10, 280, 250).astype(bf16)

    t1 = fc1_w.astype(f32).reshape(320, 50, 5, 5).transpose(2, 3, 1, 0)
    f1 = t1.reshape(5, 250, 320).astype(bf16)

    f2 = jnp.pad(fc2_w.astype(f32).T, ((0, 0), (0, 118))).astype(bf16)

    bb1 = jnp.tile(conv1_b.astype(f32), 14)[None]
    bb2 = jnp.tile(conv2_b.astype(f32), 5)[None]
    fb1 = fc1_b.astype(f32)[None]
    fb2 = jnp.pad(fc2_b.astype(f32), (0, 118))[None]
    return b1, b2, f1, f2, bb1, bb2, fb1, fb2


def kernel(x, conv1_w, conv1_b, conv2_w, conv2_b, fc1_w, fc1_b, fc2_w, fc2_b):
    tables = _build_tables(conv1_w, conv1_b, conv2_w, conv2_b,
                           fc1_w, fc1_b, fc2_w, fc2_b)
    B = x.shape[0]
    x4 = jnp.pad(x.reshape(B, 28, 28).astype(jnp.float32),
                 ((0, 0), (2, 2), (2, 2))).reshape(B, 8, 128)
    out = pl.pallas_call(
        _fused_kernel,
        out_shape=jax.ShapeDtypeStruct((B, 10), jnp.float32),
        grid=(B // _TB,),
        in_specs=[
            pl.BlockSpec((_TB, 8, 128), lambda i: (i, 0, 0)),
            pl.BlockSpec((8, 256, 280), lambda i: (0, 0, 0)),
            pl.BlockSpec((10, 280, 250), lambda i: (0, 0, 0)),
            pl.BlockSpec((5, 250, 320), lambda i: (0, 0, 0)),
            pl.BlockSpec((320, 128), lambda i: (0, 0)),
            pl.BlockSpec((1, 280), lambda i: (0, 0)),
            pl.BlockSpec((1, 250), lambda i: (0, 0)),
            pl.BlockSpec((1, 320), lambda i: (0, 0)),
            pl.BlockSpec((1, 128), lambda i: (0, 0)),
        ],
        out_specs=pl.BlockSpec((_TB, 10), lambda i: (i, 0)),
        compiler_params=pltpu.CompilerParams(dimension_semantics=("parallel",)),
    )(x4, *tables)
    return out


# speedup vs baseline: 4.4336x; 1.0309x over previous
import numpy as np

import jax
import jax.numpy as jnp
from jax.experimental import pallas as pl
from jax.experimental.pallas import tpu as pltpu

_TB = 256


def _fused_kernel(x_ref, b1_ref, b2_ref, f1_ref, f2_ref,
                  bb1_ref, bb2_ref, fb1_ref, fb2_ref, o_ref):
    tb = x_ref.shape[0]

    xv = x_ref[...].reshape(tb * 8, 128)
    xs = jnp.roll(xv, -1, axis=0)
    a1 = jnp.concatenate([xv, xs], axis=-1).astype(jnp.bfloat16)

    acc1 = [[None, None] for _ in range(4)]
    for r in range(4):
        for p in range(2):
            acc1[r][p] = jnp.dot(a1, b1_ref[2 * r + p],
                                 preferred_element_type=jnp.float32
                                 ).astype(jnp.bfloat16)
    zero = jnp.bfloat16(0.0)
    hpe = jnp.maximum(
        jnp.maximum(jnp.maximum(acc1[0][0], acc1[0][1]),
                    jnp.maximum(acc1[1][0], acc1[1][1])) + bb1_ref[...], zero
    ).reshape(tb, 8, 280)
    hpo = jnp.maximum(
        jnp.maximum(jnp.maximum(acc1[2][0], acc1[2][1]),
                    jnp.maximum(acc1[3][0], acc1[3][1])) + bb1_ref[...], zero
    ).reshape(tb, 8, 280)

    acc2 = [[None, None] for _ in range(2)]
    for q in range(2):
        for kh in range(5):
            src = hpe if (q + kh) % 2 == 0 else hpo
            t = (q + kh) // 2
            a = src[:, t:t + 5, :].reshape(tb * 5, 280)
            for p in range(2):
                d = jnp.dot(a, b2_ref[2 * kh + p],
                            preferred_element_type=jnp.float32)
                acc2[q][p] = d if acc2[q][p] is None else acc2[q][p] + d
    hp2 = jnp.maximum(
        jnp.maximum(jnp.maximum(acc2[0][0], acc2[0][1]),
                    jnp.maximum(acc2[1][0], acc2[1][1])) + bb2_ref[...], 0.0
    ).astype(jnp.bfloat16).reshape(tb, 5, 250)

    acc = None
    for h in range(5):
        d = jnp.dot(hp2[:, h, :], f1_ref[h], preferred_element_type=jnp.float32)
        acc = d if acc is None else acc + d
    hfc = jnp.maximum(acc + fb1_ref[...], 0.0).astype(jnp.bfloat16)
    o_ref[...] = (jnp.dot(hfc, f2_ref[...],
                          preferred_element_type=jnp.float32)
                  + fb2_ref[...])[:, :10]


def _band_select(ow_count, w_count):
    s = np.zeros((5, 2, w_count, ow_count), np.float32)
    for kw in range(5):
        for p in range(2):
            for ow2 in range(ow_count):
                s[kw, p, 2 * ow2 + p + kw, ow2] = 1.0
    return s


def _conv1_row_index():
    idx = np.full((4, 256), 160, np.int32)
    for r in range(4):
        for s in range(2):
            for t in range(4):
                kh = 4 * s + t - r
                if 0 <= kh <= 4:
                    for wp in range(32):
                        idx[r, s * 128 + t * 32 + wp] = kh * 32 + wp
    return idx


def _build_tables(conv1_w, conv1_b, conv2_w, conv2_b,
                  fc1_w, fc1_b, fc2_w, fc2_b):
    f32 = jnp.float32
    bf16 = jnp.bfloat16
    w1 = conv1_w[:, 0].transpose(1, 2, 0).astype(f32)
    w2 = conv2_w.transpose(2, 3, 1, 0).astype(f32)

    s1 = jnp.asarray(_band_select(14, 32))
    b1k = jnp.einsum('akc,kpwm->pawmc', w1, s1)
    b1k = b1k.reshape(2, 160, 280)
    b1aug = jnp.concatenate([b1k, jnp.zeros((2, 1, 280), f32)], axis=1)
    idx = jnp.asarray(_conv1_row_index())
    b1 = b1aug[:, idx, :].transpose(1, 0, 2, 3)
    b1 = b1.reshape(8, 256, 280).astype(bf16)

    s2 = jnp.asarray(_band_select(5, 14))
    b2 = jnp.einsum('akco,kpwm->apwcmo', w2, s2)
    b2 = b2.reshape(10, 280, 250).astype(bf16)

    t1 = fc1_w.astype(f32).reshape(320, 50, 5, 5).transpose(2, 3, 1, 0)
    f1 = t1.reshape(5, 250, 320).astype(bf16)

    f2 = jnp.pad(fc2_w.astype(f32).T, ((0, 0), (0, 118))).astype(bf16)

    bb1 = jnp.tile(conv1_b, 14)[None].astype(bf16)
    bb2 = jnp.tile(conv2_b.astype(f32), 5)[None]
    fb1 = fc1_b.astype(f32)[None]
    fb2 = jnp.pad(fc2_b.astype(f32), (0, 118))[None]
    return b1, b2, f1, f2, bb1, bb2, fb1, fb2


def kernel(x, conv1_w, conv1_b, conv2_w, conv2_b, fc1_w, fc1_b, fc2_w, fc2_b):
    tables = _build_tables(conv1_w, conv1_b, conv2_w, conv2_b,
                           fc1_w, fc1_b, fc2_w, fc2_b)
    B = x.shape[0]
    x4 = jnp.pad(x.reshape(B, 28, 28).astype(jnp.float32),
                 ((0, 0), (2, 2), (2, 2))).reshape(B, 8, 128)
    out = pl.pallas_call(
        _fused_kernel,
        out_shape=jax.ShapeDtypeStruct((B, 10), jnp.float32),
        grid=(B // _TB,),
        in_specs=[
            pl.BlockSpec((_TB, 8, 128), lambda i: (i, 0, 0)),
            pl.BlockSpec((8, 256, 280), lambda i: (0, 0, 0)),
            pl.BlockSpec((10, 280, 250), lambda i: (0, 0, 0)),
            pl.BlockSpec((5, 250, 320), lambda i: (0, 0, 0)),
            pl.BlockSpec((320, 128), lambda i: (0, 0)),
            pl.BlockSpec((1, 280), lambda i: (0, 0)),
            pl.BlockSpec((1, 250), lambda i: (0, 0)),
            pl.BlockSpec((1, 320), lambda i: (0, 0)),
            pl.BlockSpec((1, 128), lambda i: (0, 0)),
        ],
        out_specs=pl.BlockSpec((_TB, 10), lambda i: (i, 0)),
        compiler_params=pltpu.CompilerParams(dimension_semantics=("parallel",)),
    )(x4, *tables)
    return out
